# Initial kernel scaffold; baseline (speedup 1.0000x reference)
import jax
import jax.numpy as jnp
from jax.experimental import pallas as pl


def kernel(x, W1, b1, W2, b2, expert_W, expert_b):
    raise NotImplementedError("write your pallas kernel here")


# speedup vs baseline: 5.5791x; 5.5791x over previous
import functools

import jax
import jax.numpy as jnp
from jax.experimental import pallas as pl
from jax.experimental.pallas import tpu as pltpu

HIDDEN = 2048
NUM_EXPERTS = 16
TOP_K = 2


def _router_kernel(x0_ref, W1_ref, b1_ref, W2_ref, b2_ref, eb_ref,
                   idx_ref, w_ref, bc_ref):
    pooled = jnp.mean(x0_ref[...], axis=0, keepdims=True)
    h = jnp.dot(pooled, W1_ref[...], preferred_element_type=jnp.float32)
    h = h + b1_ref[...]
    h = h * jax.nn.sigmoid(h)
    logits = jnp.dot(h, W2_ref[...], preferred_element_type=jnp.float32)
    logits = logits + b2_ref[...]

    iota = jax.lax.broadcasted_iota(jnp.int32, (1, NUM_EXPERTS), 1)
    m0 = jnp.max(logits)
    i0 = jnp.min(jnp.where(logits == m0, iota, NUM_EXPERTS))
    masked = jnp.where(iota == i0, -jnp.inf, logits)
    m1 = jnp.max(masked)
    i1 = jnp.min(jnp.where(masked == m1, iota, NUM_EXPERTS))
    w0 = 1.0 / (1.0 + jnp.exp(m1 - m0))
    w1 = 1.0 - w0

    idx_ref[0] = i0
    idx_ref[1] = i1
    w_ref[0] = w0
    w_ref[1] = w1

    wvec = jnp.where(iota == i0, w0, 0.0) + jnp.where(iota == i1, w1, 0.0)
    bc_ref[...] = jnp.dot(wvec, eb_ref[...], preferred_element_type=jnp.float32)


def _combine_kernel(idx_ref, w_ref, w0_ref, w1_ref, wc_ref):
    wc_ref[...] = w_ref[0] * w0_ref[0] + w_ref[1] * w1_ref[0]


def _gemm_kernel(x_ref, wc_ref, bc_ref, out_ref):
    out_ref[...] = jnp.dot(x_ref[...], wc_ref[...],
                           preferred_element_type=jnp.float32) + bc_ref[...]


@jax.jit
def kernel(x, W1, b1, W2, b2, expert_W, expert_b):
    B, S, H = x.shape
    E = expert_W.shape[0]

    idx, w, bc = pl.pallas_call(
        _router_kernel,
        out_shape=[
            jax.ShapeDtypeStruct((TOP_K,), jnp.int32),
            jax.ShapeDtypeStruct((TOP_K,), jnp.float32),
            jax.ShapeDtypeStruct((1, H), jnp.float32),
        ],
        out_specs=[
            pl.BlockSpec(memory_space=pltpu.SMEM),
            pl.BlockSpec(memory_space=pltpu.SMEM),
            pl.BlockSpec(memory_space=pltpu.VMEM),
        ],
    )(x[0], W1, b1.reshape(1, -1), W2, b2.reshape(1, -1), expert_b)

    HT = 8
    wc = pl.pallas_call(
        _combine_kernel,
        grid_spec=pltpu.PrefetchScalarGridSpec(
            num_scalar_prefetch=2,
            grid=(HT,),
            in_specs=[
                pl.BlockSpec((1, H // HT, H),
                             lambda h, idx, w: (idx[0], h, 0)),
                pl.BlockSpec((1, H // HT, H),
                             lambda h, idx, w: (idx[1], h, 0)),
            ],
            out_specs=pl.BlockSpec((H // HT, H), lambda h, idx, w: (h, 0)),
        ),
        out_shape=jax.ShapeDtypeStruct((H, H), jnp.float32),
    )(idx, w, expert_W, expert_W)

    M = B * S
    MT = 512
    x2 = x.reshape(M, H)
    y = pl.pallas_call(
        _gemm_kernel,
        grid=(M // MT,),
        in_specs=[
            pl.BlockSpec((MT, H), lambda m: (m, 0)),
            pl.BlockSpec((H, H), lambda m: (0, 0)),
            pl.BlockSpec((1, H), lambda m: (0, 0)),
        ],
        out_specs=pl.BlockSpec((MT, H), lambda m: (m, 0)),
        out_shape=jax.ShapeDtypeStruct((M, H), jnp.float32),
    )(x2, wc, bc)

    return y.reshape(B, S, H)


# speedup vs baseline: 6.5908x; 1.1813x over previous
import functools

import jax
import jax.numpy as jnp
from jax.experimental import pallas as pl
from jax.experimental.pallas import tpu as pltpu

HIDDEN = 2048
NUM_EXPERTS = 16
TOP_K = 2


def _router_kernel(x_ref, W1_ref, b1_ref, W2_ref, b2_ref, eb_ref,
                   idx_ref, w_ref, bc_ref):
    pooled = jnp.mean(x_ref[0], axis=0, keepdims=True)
    h = jnp.dot(pooled, W1_ref[...], preferred_element_type=jnp.float32)
    h = h + b1_ref[...]
    h = h * jax.nn.sigmoid(h)
    logits = jnp.dot(h, W2_ref[...], preferred_element_type=jnp.float32)
    logits = logits + b2_ref[...]

    iota = jax.lax.broadcasted_iota(jnp.int32, (1, NUM_EXPERTS), 1)
    m0 = jnp.max(logits)
    i0 = jnp.min(jnp.where(logits == m0, iota, NUM_EXPERTS))
    masked = jnp.where(iota == i0, -jnp.inf, logits)
    m1 = jnp.max(masked)
    i1 = jnp.min(jnp.where(masked == m1, iota, NUM_EXPERTS))
    w0 = 1.0 / (1.0 + jnp.exp(m1 - m0))
    w1 = 1.0 - w0

    idx_ref[0] = i0
    idx_ref[1] = i1
    w_ref[0] = w0
    w_ref[1] = w1

    wvec = jnp.where(iota == i0, w0, 0.0) + jnp.where(iota == i1, w1, 0.0)
    bc_ref[...] = jnp.dot(wvec, eb_ref[...], preferred_element_type=jnp.float32)


def _moe_gemm_kernel(idx_ref, w_ref, x_ref, w0_ref, w1_ref, bc_ref,
                     out_ref, wc_ref):
    @pl.when(jnp.logical_and(pl.program_id(0) == 0, pl.program_id(1) == 0))
    def _combine():
        wc = w_ref[0] * w0_ref[0] + w_ref[1] * w1_ref[0]
        wc_ref[...] = wc.astype(jnp.bfloat16)

    acc = jnp.dot(x_ref[0].astype(jnp.bfloat16), wc_ref[...],
                  preferred_element_type=jnp.float32)
    out_ref[0] = acc + bc_ref[...]


@jax.jit
def kernel(x, W1, b1, W2, b2, expert_W, expert_b):
    B, S, H = x.shape

    idx, w, bc = pl.pallas_call(
        _router_kernel,
        grid=(1,),
        in_specs=[
            pl.BlockSpec((1, S, H), lambda i: (0, 0, 0)),
            pl.BlockSpec((H, H // 2), lambda i: (0, 0)),
            pl.BlockSpec((1, H // 2), lambda i: (0, 0)),
            pl.BlockSpec((H // 2, NUM_EXPERTS), lambda i: (0, 0)),
            pl.BlockSpec((1, NUM_EXPERTS), lambda i: (0, 0)),
            pl.BlockSpec((NUM_EXPERTS, H), lambda i: (0, 0)),
        ],
        out_shape=[
            jax.ShapeDtypeStruct((TOP_K,), jnp.int32),
            jax.ShapeDtypeStruct((TOP_K,), jnp.float32),
            jax.ShapeDtypeStruct((1, H), jnp.float32),
        ],
        out_specs=[
            pl.BlockSpec(memory_space=pltpu.SMEM),
            pl.BlockSpec(memory_space=pltpu.SMEM),
            pl.BlockSpec((1, H), lambda i: (0, 0)),
        ],
    )(x, W1, b1.reshape(1, -1), W2, b2.reshape(1, -1), expert_b)

    MTS = 512
    y = pl.pallas_call(
        _moe_gemm_kernel,
        grid_spec=pltpu.PrefetchScalarGridSpec(
            num_scalar_prefetch=2,
            grid=(B, S // MTS),
            in_specs=[
                pl.BlockSpec((1, MTS, H), lambda b, s, idx, w: (b, s, 0)),
                pl.BlockSpec((1, H, H), lambda b, s, idx, w: (idx[0], 0, 0)),
                pl.BlockSpec((1, H, H), lambda b, s, idx, w: (idx[1], 0, 0)),
                pl.BlockSpec((1, H), lambda b, s, idx, w: (0, 0)),
            ],
            out_specs=pl.BlockSpec((1, MTS, H), lambda b, s, idx, w: (b, s, 0)),
            scratch_shapes=[pltpu.VMEM((H, H), jnp.bfloat16)],
        ),
        out_shape=jax.ShapeDtypeStruct((B, S, H), jnp.float32),
        compiler_params=pltpu.CompilerParams(
            vmem_limit_bytes=100 * 1024 * 1024,
        ),
    )(idx, w, x, expert_W, expert_W, bc)

    return y
